# baseline (speedup 1.0000x reference)
import functools

import jax
import jax.numpy as jnp
from jax import lax
from jax.experimental import pallas as pl
from jax.experimental.pallas import tpu as pltpu
from jax.experimental.pallas import tpu_sc as plsc

VOCAB = 1000000
EMB = 64
NW = 32
CHUNK = 128


def _emb_kernel(n_rows):
    per_w = n_rows // NW
    nch = per_w // CHUNK
    mesh = plsc.VectorSubcoreMesh(core_axis_name="c", subcore_axis_name="s")

    @functools.partial(
        pl.kernel,
        out_type=jax.ShapeDtypeStruct((n_rows, EMB), jnp.float32),
        mesh=mesh,
        scratch_types=[
            pltpu.VMEM((nch, CHUNK), jnp.int32),
            pltpu.VMEM((CHUNK, EMB), jnp.float32),
            pltpu.SemaphoreType.DMA,
        ],
        compiler_params=pltpu.CompilerParams(use_tc_tiling_on_sc=False),
    )
    def k(idx_hbm, table_hbm, out_hbm, idx_v, rows_v, gsem):
        wid = lax.axis_index("s") * 2 + lax.axis_index("c")
        base = wid * per_w
        pltpu.sync_copy(idx_hbm.at[wid], idx_v)

        def body(j, carry):
            pltpu.async_copy(table_hbm.at[idx_v.at[j]], rows_v, gsem).wait()
            pltpu.sync_copy(rows_v, out_hbm.at[pl.ds(base + j * CHUNK, CHUNK)])
            return carry

        lax.fori_loop(0, nch, body, 0, unroll=False)

    return k


def kernel(inputs, table):
    b, l = inputs.shape
    n_rows = b * l
    idx = inputs.reshape(NW, n_rows // (NW * CHUNK), CHUNK).astype(jnp.int32)
    out = _emb_kernel(n_rows)(idx, table)
    return out.reshape(b, l, EMB)


# speedup vs baseline: 1.1101x; 1.1101x over previous
import functools

import jax
import jax.numpy as jnp
from jax import lax
from jax.experimental import pallas as pl
from jax.experimental.pallas import tpu as pltpu
from jax.experimental.pallas import tpu_sc as plsc

VOCAB = 1000000
EMB = 64
NW = 32
CHUNK = 128
NBUF = 8


def _emb_kernel(n_rows):
    per_w = n_rows // NW
    nch = per_w // CHUNK
    ngroups = nch // NBUF
    mesh = plsc.VectorSubcoreMesh(core_axis_name="c", subcore_axis_name="s")

    @functools.partial(
        pl.kernel,
        out_type=jax.ShapeDtypeStruct((n_rows, EMB), jnp.float32),
        mesh=mesh,
        scratch_types=[
            pltpu.VMEM((nch, CHUNK), jnp.int32),
            [pltpu.VMEM((CHUNK, EMB), jnp.float32) for _ in range(NBUF)],
            [pltpu.SemaphoreType.DMA for _ in range(NBUF)],
            [pltpu.SemaphoreType.DMA for _ in range(NBUF)],
        ],
        compiler_params=pltpu.CompilerParams(use_tc_tiling_on_sc=False),
    )
    def k(idx_hbm, table_hbm, out_hbm, idx_v, bufs, gsems, ssems):
        wid = lax.axis_index("s") * 2 + lax.axis_index("c")
        base = wid * per_w
        pltpu.sync_copy(idx_hbm.at[wid], idx_v)

        def gather(j, b):
            pltpu.async_copy(table_hbm.at[idx_v.at[j]], bufs[b], gsems[b])

        def wait_gather(j, b):
            pltpu.make_async_copy(
                table_hbm.at[idx_v.at[j]], bufs[b], gsems[b]).wait()

        def store(j, b):
            pltpu.async_copy(
                bufs[b], out_hbm.at[pl.ds(base + j * CHUNK, CHUNK)], ssems[b])

        def wait_store(b):
            pltpu.make_async_copy(
                bufs[b], out_hbm.at[pl.ds(base, CHUNK)], ssems[b]).wait()

        for b in range(NBUF):
            gather(b, b)

        def body(g, carry):
            for b in range(NBUF):
                j = g * NBUF + b
                wait_gather(j, b)
                store(j, b)
            for b in range(NBUF):
                j2 = (g + 1) * NBUF + b
                wait_store(b)
                gather(j2, b)
            return carry

        lax.fori_loop(0, ngroups - 1, body, 0, unroll=False)

        for b in range(NBUF):
            j = (ngroups - 1) * NBUF + b
            wait_gather(j, b)
            store(j, b)
        for b in range(NBUF):
            wait_store(b)

    return k


def kernel(inputs, table):
    b, l = inputs.shape
    n_rows = b * l
    idx = inputs.reshape(NW, n_rows // (NW * CHUNK), CHUNK).astype(jnp.int32)
    out = _emb_kernel(n_rows)(idx, table)
    return out.reshape(b, l, EMB)


# speedup vs baseline: 1.6524x; 1.4885x over previous
import functools

import jax
import jax.numpy as jnp
from jax import lax
from jax.experimental import pallas as pl
from jax.experimental.pallas import tpu as pltpu
from jax.experimental.pallas import tpu_sc as plsc

VOCAB = 1000000
EMB = 64
B = 4096
L = 200
NC = 2
NS = 16
PLANES_PER_SC = EMB // NC
LMAX = -(-L // NS)


def _plane_kernel():
    mesh = plsc.VectorSubcoreMesh(core_axis_name="c", subcore_axis_name="s")

    @functools.partial(
        pl.kernel,
        out_type=jax.ShapeDtypeStruct((L, EMB, B), jnp.float32),
        mesh=mesh,
        scratch_types=[
            pltpu.VMEM_SHARED((VOCAB,), jnp.float32),
            [pltpu.VMEM((B,), jnp.int32) for _ in range(LMAX)],
            pltpu.VMEM((B,), jnp.float32),
            pltpu.SemaphoreType.DMA,
        ],
    )
    def k(idx_hbm, table_hbm, out_hbm, plane_sh, idx_refs, buf, gsem):
        c = lax.axis_index("c")
        s = lax.axis_index("s")

        for kk in range(LMAX):
            l = s + NS * kk

            @pl.when(l < L)
            def _(l=l, kk=kk):
                pltpu.sync_copy(idx_hbm.at[l], idx_refs[kk])

        def per_plane(p, carry):
            f = c * PLANES_PER_SC + p

            @pl.when(s == 0)
            def _():
                pltpu.sync_copy(table_hbm.at[f], plane_sh)

            plsc.subcore_barrier()

            for kk in range(LMAX):
                l = s + NS * kk

                @pl.when(l < L)
                def _(l=l, kk=kk):
                    pltpu.async_copy(
                        plane_sh.at[idx_refs[kk]], buf, gsem).wait()
                    pltpu.sync_copy(buf, out_hbm.at[l, f])

            plsc.subcore_barrier()
            return carry

        lax.fori_loop(0, PLANES_PER_SC, per_plane, 0, unroll=False)

    return k


def kernel(inputs, table):
    table_t = jnp.swapaxes(table, 0, 1)
    idx_t = jnp.swapaxes(inputs, 0, 1).astype(jnp.int32)
    out = _plane_kernel()(idx_t, table_t)
    return jnp.transpose(out, (2, 0, 1))


# speedup vs baseline: 1.9851x; 1.2013x over previous
import functools

import jax
import jax.numpy as jnp
from jax import lax
from jax.experimental import pallas as pl
from jax.experimental.pallas import tpu as pltpu
from jax.experimental.pallas import tpu_sc as plsc

VOCAB = 1000000
EMB = 64
B = 4096
L = 200
NC = 2
NS = 16
PLANES_PER_SC = EMB // NC
H = B // 2
UNITS = L * 2 // NS
NB = 4
NSTAGE = 12
CHUNK = 83328
TAILOFF = NSTAGE * CHUNK
VPAD = TAILOFF + 128


def _plane_kernel():
    mesh = plsc.VectorSubcoreMesh(core_axis_name="c", subcore_axis_name="s")

    @functools.partial(
        pl.kernel,
        out_type=jax.ShapeDtypeStruct((L, EMB, B), jnp.float32),
        mesh=mesh,
        scratch_types=[
            [pltpu.VMEM_SHARED((VPAD,), jnp.float32) for _ in range(1)],
            [pltpu.VMEM((H,), jnp.int32) for _ in range(UNITS)],
            [pltpu.VMEM((H,), jnp.float32) for _ in range(NB)],
            [pltpu.SemaphoreType.DMA for _ in range(1)],
            [pltpu.SemaphoreType.DMA for _ in range(NB)],
            [pltpu.SemaphoreType.DMA for _ in range(NB)],
        ],
    )
    def k(idx_hbm, table_hbm, tail_hbm, out_hbm, planes, idx_refs, bufs,
          psems, gsems, ssems):
        c = lax.axis_index("c")
        s = lax.axis_index("s")

        def unit_lh(u):
            uid = s + NS * u
            return uid // 2, (uid % 2) * H

        for u in range(UNITS):
            l, h = unit_lh(u)
            pltpu.sync_copy(idx_hbm.at[l].at[pl.dslice(h, H)], idx_refs[u])

        def stage(f, pb):
            base = s * CHUNK

            @pl.when(s < NSTAGE)
            def _():
                pltpu.async_copy(
                    table_hbm.at[f].at[pl.dslice(base, CHUNK)],
                    planes[pb].at[pl.dslice(base, CHUNK)],
                    psems[pb],
                )

            @pl.when(s == NSTAGE)
            def _():
                pltpu.async_copy(
                    tail_hbm.at[f],
                    planes[pb].at[pl.dslice(TAILOFF, 128)],
                    psems[pb],
                )

        def wait_stage(pb):
            @pl.when(s < NSTAGE)
            def _():
                pltpu.make_async_copy(
                    table_hbm.at[0].at[pl.dslice(0, CHUNK)],
                    planes[pb].at[pl.dslice(0, CHUNK)],
                    psems[pb],
                ).wait()

            @pl.when(s == NSTAGE)
            def _():
                pltpu.make_async_copy(
                    tail_hbm.at[0],
                    planes[pb].at[pl.dslice(TAILOFF, 128)],
                    psems[pb],
                ).wait()

        def sweep(f, pb):
            def gather(u, gb):
                pltpu.async_copy(
                    planes[pb].at[idx_refs[u]], bufs[gb], gsems[gb])

            def wait_gather(u, gb):
                pltpu.make_async_copy(
                    planes[pb].at[idx_refs[u]], bufs[gb], gsems[gb]).wait()

            def wait_store(gb):
                pltpu.make_async_copy(
                    bufs[gb], out_hbm.at[0].at[0].at[pl.dslice(0, H)],
                    ssems[gb]).wait()

            for u in range(NB):
                gather(u, u)
            for u in range(UNITS):
                gb = u % NB
                l, h = unit_lh(u)
                wait_gather(u, gb)
                pltpu.async_copy(
                    bufs[gb], out_hbm.at[l].at[f].at[pl.dslice(h, H)], ssems[gb])
                if u + NB < UNITS:
                    wait_store(gb)
                    gather(u + NB, gb)
            for u in range(UNITS - NB, UNITS):
                wait_store(u % NB)

        f0 = c * PLANES_PER_SC
        stage(f0, 0)
        wait_stage(0)
        plsc.subcore_barrier()

        def body(p, carry):
            sweep(f0 + p, 0)
            plsc.subcore_barrier()

            @pl.when(p < PLANES_PER_SC - 1)
            def _():
                stage(f0 + p + 1, 0)
                wait_stage(0)

            plsc.subcore_barrier()
            return carry

        lax.fori_loop(0, PLANES_PER_SC, body, 0, unroll=False)

    return k


def kernel(inputs, table):
    table_t = jnp.swapaxes(table, 0, 1)
    idx_t = jnp.swapaxes(inputs, 0, 1).astype(jnp.int32)
    tail = jnp.pad(table_t[:, TAILOFF:], ((0, 0), (0, VPAD - VOCAB)))
    out = _plane_kernel()(idx_t, table_t, tail)
    return jnp.transpose(out, (2, 0, 1))


# speedup vs baseline: 2.0009x; 1.0079x over previous
import functools

import jax
import jax.numpy as jnp
from jax import lax
from jax.experimental import pallas as pl
from jax.experimental.pallas import tpu as pltpu
from jax.experimental.pallas import tpu_sc as plsc

VOCAB = 1000000
EMB = 64
B = 4096
L = 200
NC = 2
NS = 16
PLANES_PER_SC = EMB // NC
H = B // 2
UNITS = L * 2 // NS
NB = 4
CHUNK = 62464
XTRA = 999936 - NS * CHUNK
TAILOFF = 999936
VPAD = TAILOFF + 128


def _plane_kernel():
    mesh = plsc.VectorSubcoreMesh(core_axis_name="c", subcore_axis_name="s")

    @functools.partial(
        pl.kernel,
        out_type=jax.ShapeDtypeStruct((L, EMB, B), jnp.float32),
        mesh=mesh,
        scratch_types=[
            [pltpu.VMEM_SHARED((VPAD,), jnp.float32) for _ in range(1)],
            [pltpu.VMEM((H,), jnp.int32) for _ in range(UNITS)],
            [pltpu.VMEM((H,), jnp.float32) for _ in range(NB)],
            [pltpu.SemaphoreType.DMA for _ in range(1)],
            [pltpu.SemaphoreType.DMA for _ in range(NB)],
            [pltpu.SemaphoreType.DMA for _ in range(NB)],
        ],
    )
    def k(idx_hbm, table_hbm, tail_hbm, out_hbm, planes, idx_refs, bufs,
          psems, gsems, ssems):
        c = lax.axis_index("c")
        s = lax.axis_index("s")

        def unit_lh(u):
            uid = s + NS * u
            return uid // 2, (uid % 2) * H

        for u in range(UNITS):
            l, h = unit_lh(u)
            pltpu.sync_copy(idx_hbm.at[l].at[pl.dslice(h, H)], idx_refs[u])

        def stage(f, pb):
            pltpu.async_copy(
                table_hbm.at[f].at[pl.dslice(s * CHUNK, CHUNK)],
                planes[pb].at[pl.dslice(s * CHUNK, CHUNK)],
                psems[pb],
            )

            @pl.when(s < 4)
            def _():
                pltpu.async_copy(
                    table_hbm.at[f].at[pl.dslice(NS * CHUNK + 128 * s, 128)],
                    planes[pb].at[pl.dslice(NS * CHUNK + 128 * s, 128)],
                    psems[pb],
                )

            @pl.when(s == 4)
            def _():
                pltpu.async_copy(
                    tail_hbm.at[f],
                    planes[pb].at[pl.dslice(TAILOFF, 128)],
                    psems[pb],
                )

        def wait_stage(pb):
            pltpu.make_async_copy(
                table_hbm.at[0].at[pl.dslice(0, CHUNK)],
                planes[pb].at[pl.dslice(0, CHUNK)],
                psems[pb],
            ).wait()

            @pl.when((s < 4) | (s == 4))
            def _():
                pltpu.make_async_copy(
                    tail_hbm.at[0],
                    planes[pb].at[pl.dslice(TAILOFF, 128)],
                    psems[pb],
                ).wait()

        def sweep_main(f, pb):
            def gather(u, gb):
                pltpu.async_copy(
                    planes[pb].at[idx_refs[u]], bufs[gb], gsems[gb])

            def wait_gather(u, gb):
                pltpu.make_async_copy(
                    planes[pb].at[idx_refs[u]], bufs[gb], gsems[gb]).wait()

            def wait_store(gb):
                pltpu.make_async_copy(
                    bufs[gb], out_hbm.at[0].at[0].at[pl.dslice(0, H)],
                    ssems[gb]).wait()

            for u in range(NB):
                gather(u, u)
            for u in range(UNITS):
                gb = u % NB
                l, h = unit_lh(u)
                wait_gather(u, gb)
                pltpu.async_copy(
                    bufs[gb], out_hbm.at[l].at[f].at[pl.dslice(h, H)], ssems[gb])
                if u + NB < UNITS:
                    wait_store(gb)
                    gather(u + NB, gb)
        def drain_stores():
            for u in range(UNITS - NB, UNITS):
                pltpu.make_async_copy(
                    bufs[u % NB], out_hbm.at[0].at[0].at[pl.dslice(0, H)],
                    ssems[u % NB]).wait()

        f0 = c * PLANES_PER_SC
        stage(f0, 0)
        wait_stage(0)
        plsc.subcore_barrier()

        def body(p, carry):
            sweep_main(f0 + p, 0)
            plsc.subcore_barrier()

            @pl.when(p < PLANES_PER_SC - 1)
            def _():
                stage(f0 + p + 1, 0)

            drain_stores()

            @pl.when(p < PLANES_PER_SC - 1)
            def _():
                wait_stage(0)

            plsc.subcore_barrier()
            return carry

        lax.fori_loop(0, PLANES_PER_SC, body, 0, unroll=False)

    return k


def kernel(inputs, table):
    table_t = jnp.swapaxes(table, 0, 1)
    idx_t = jnp.swapaxes(inputs, 0, 1).astype(jnp.int32)
    tail = jnp.pad(table_t[:, TAILOFF:], ((0, 0), (0, VPAD - VOCAB)))
    out = _plane_kernel()(idx_t, table_t, tail)
    return jnp.transpose(out, (2, 0, 1))
